# baseline (speedup 1.0000x reference)
import functools

import jax
import jax.numpy as jnp
from jax import lax
from jax.experimental import pallas as pl
from jax.experimental.pallas import tpu as pltpu
from jax.experimental.pallas import tpu_sc as plsc

BATCH = 16384
FIELDS = 26
FEATURES = 32

NUM_CORES = 2
NUM_SUBCORES = 16
NUM_WORKERS = NUM_CORES * NUM_SUBCORES
BTILES = BATCH // 128
BT_PER_WORKER = BTILES // NUM_WORKERS
NUNITS = FIELDS
ROWS = BT_PER_WORKER * 128
NBUF = 2


@functools.partial(
    pl.kernel,
    mesh=plsc.VectorSubcoreMesh(core_axis_name="c", subcore_axis_name="s"),
    compiler_params=pltpu.CompilerParams(use_tc_tiling_on_sc=False),
    out_type=jax.ShapeDtypeStruct((FIELDS, BATCH, FEATURES), jnp.float32),
    scratch_types=[
        pltpu.VMEM((NBUF, ROWS), jnp.int32),
        pltpu.VMEM((NBUF, ROWS, FEATURES), jnp.float32),
        pltpu.VMEM((ROWS, FEATURES), jnp.float32),
        pltpu.SemaphoreType.DMA((NBUF,)),
        pltpu.SemaphoreType.DMA((NBUF,)),
        pltpu.SemaphoreType.DMA((NBUF,)),
    ],
)
def _sc_gather(idxt_hbm, table_hbm, out_hbm, list_v, rows_v, dummy_v,
               isem, gsem, wsem):
    wid = lax.axis_index("s") * NUM_CORES + lax.axis_index("c")
    t0 = wid * BT_PER_WORKER

    def start_idx(u, b):
        return pltpu.async_copy(
            idxt_hbm.at[pl.ds(u * BATCH + t0 * 128, ROWS)],
            list_v.at[b],
            isem.at[b],
        )

    def start_gather(b):
        return pltpu.async_copy(
            table_hbm.at[list_v.at[b]], rows_v.at[b], gsem.at[b]
        )

    def start_write(u, b):
        return pltpu.async_copy(
            rows_v.at[b],
            out_hbm.at[u, pl.ds(t0 * 128, ROWS)],
            wsem.at[b],
        )

    def drain_write(b):
        pltpu.make_async_copy(
            dummy_v, out_hbm.at[0, pl.ds(0, ROWS)], wsem.at[b]
        ).wait()

    def scale_list(b):
        for g in range(ROWS // 16):
            v = list_v[b, pl.ds(g * 16, 16)]
            list_v[b, pl.ds(g * 16, 16)] = v * 4

    def body(i, carry):
        u0 = i * NBUF
        idx_dmas = []
        for b in range(NBUF):
            @pl.when(i > 0)
            def _():
                drain_write(b)
            idx_dmas.append(start_idx(u0 + b, b))
        gathers = []
        for b in range(NBUF):
            idx_dmas[b].wait()
            scale_list(b)
            gathers.append(start_gather(b))
        for b in range(NBUF):
            gathers[b].wait()
            start_write(u0 + b, b)
        return carry

    lax.fori_loop(0, NUNITS // NBUF, body, 0)
    for b in range(NBUF):
        drain_write(b)


def kernel(inputs, embedding):
    idxt = inputs.T.reshape(-1)
    emb_pad = jnp.concatenate(
        [embedding, jnp.zeros((1000000, 96), jnp.float32)], axis=1
    ).reshape(4000000, 32)
    out3 = _sc_gather(idxt, emb_pad)
    return jnp.swapaxes(out3, 0, 1)
